# baseline (speedup 1.0000x reference)
import jax
import jax.numpy as jnp
from jax import lax
from jax.experimental import pallas as pl
from jax.experimental.pallas import tpu as pltpu
from jax.experimental.pallas import tpu_sc as plsc

NUM_CLASSES = 80
N_FG_OUT = 128
N_BG_OUT = 384
N_REAL = 5100
N_PAD = 5120
G = 100
G_PAD = 128

_NC = 2
_NS = 16
_PER_W = N_PAD // (_NC * _NS)
_CHUNKS = _PER_W // 16
_G_CH = 6
_PER_C = N_PAD // _NC

_CAP = _PER_C + 64
_TRASH = _PER_C
_XFER = 384
_ST_LEN = 2 * _XFER + 32
_ST_CNT = 2 * _XFER
_ST_FLAG = 2 * _XFER + 16
_MAGIC = 1234567
_SPIN = 64

_mesh = plsc.VectorSubcoreMesh(core_axis_name="c", subcore_axis_name="s")


def _fused_body(px1h, py1h, px2h, py2h, gx1h, gy1h, gx2h, gy2h, gclsh,
                ox1h, oy1h, ox2h, oy2h, oiouh, oclsh, cnth, bvh, bih, sth,
                px1, py1, px2, py2, gx1, gy1, gx2, gy2, gcls,
                bv, bi, fgloc, bgloc, tfga, tfgb, tbga, tbgb,
                cntv, callv, c1v, flagv, stagev, stagev2,
                sidxa, sidxb, sidxc, sidxd, ava, avb, avc, avd,
                selref, bvres, bires,
                ox1, oy1, ox2, oy2, oiou, ocls,
                fgsh, bgsh, cntsh, fg1sh, bg1sh, c1sh, sem):
    c = lax.axis_index("c")
    s = lax.axis_index("s")
    wid = c * _NS + s
    base = wid * _PER_W
    iot = lax.iota(jnp.int32, 16)

    @pl.when((c == 0) & (s == 0))
    def _():
        c1v[...] = jnp.zeros((16,), jnp.int32)
        pltpu.sync_copy(c1v, sth.at[pl.ds(_ST_FLAG, 16)])

    loads = [pltpu.async_copy(px1h.at[pl.ds(base, _PER_W)], px1, sem),
             pltpu.async_copy(py1h.at[pl.ds(base, _PER_W)], py1, sem),
             pltpu.async_copy(px2h.at[pl.ds(base, _PER_W)], px2, sem),
             pltpu.async_copy(py2h.at[pl.ds(base, _PER_W)], py2, sem),
             pltpu.async_copy(gx1h, gx1, sem),
             pltpu.async_copy(gy1h, gy1, sem),
             pltpu.async_copy(gx2h, gx2, sem),
             pltpu.async_copy(gy2h, gy2, sem),
             pltpu.async_copy(gclsh, gcls, sem)]
    for h in loads:
        h.wait()

    _TILE = 10
    for t in range(_CHUNKS // _TILE):
        sls = [pl.ds((t * _TILE + k) * 16, 16) for k in range(_TILE)]
        x1s = [px1[sl] for sl in sls]
        y1s = [py1[sl] for sl in sls]
        x2s = [px2[sl] for sl in sls]
        y2s = [py2[sl] for sl in sls]
        pas = [(x2s[k] - x1s[k]) * (y2s[k] - y1s[k]) for k in range(_TILE)]

        def body(gc, carry, nl=16, x1s=x1s, y1s=y1s, x2s=x2s, y2s=y2s,
                 pas=pas):
            bvs = list(carry[:_TILE])
            bis = list(carry[_TILE:])
            g0 = pl.ds(gc * 16, 16)
            a1v = gx1[g0]
            b1v = gy1[g0]
            a2v = gx2[g0]
            b2v = gy2[g0]
            gav = (a2v - a1v) * (b2v - b1v)
            jbase = gc * 16
            for l in range(nl):
                a1 = a1v[l]
                b1 = b1v[l]
                a2 = a2v[l]
                b2 = b2v[l]
                ga = gav[l]
                for k in range(_TILE):
                    w = jnp.maximum(
                        jnp.minimum(a2, x2s[k]) - jnp.maximum(a1, x1s[k]),
                        0.0)
                    h = jnp.maximum(
                        jnp.minimum(b2, y2s[k]) - jnp.maximum(b1, y1s[k]),
                        0.0)
                    inter = w * h
                    iou = inter / ((ga + pas[k]) - inter)
                    pred = iou > bvs[k]
                    bvs[k] = jnp.where(pred, iou, bvs[k])
                    bis[k] = jnp.where(pred, jbase + l, bis[k])
            return tuple(bvs) + tuple(bis)

        init = (tuple(jnp.full((16,), -1.0, jnp.float32)
                      for _ in range(_TILE))
                + tuple(jnp.zeros((16,), jnp.int32) for _ in range(_TILE)))
        carry = lax.fori_loop(0, _G_CH, body, init)
        res = body(_G_CH, carry, nl=G - _G_CH * 16)
        for k in range(_TILE):
            bv[sls[k]] = res[k]
            bi[sls[k]] = res[_TILE + k]

    hm1 = pltpu.async_copy(bv, bvh.at[pl.ds(base, _PER_W)], sem)
    hm2 = pltpu.async_copy(bi, bih.at[pl.ds(base, _PER_W)], sem)
    hm1.wait()
    hm2.wait()

    def cbody(ch, carry):
        fgoff, bgoff = carry
        vals = bv[pl.ds(ch * 16, 16)]
        gidx = iot + base + ch * 16
        real = gidx < N_REAL
        fgm = (vals >= 0.5) & real
        bgm = jnp.logical_not(vals >= 0.5) & real
        fgc = jnp.where(fgm, 1, 0)
        bgc = jnp.where(bgm, 1, 0)
        fgrank = fgoff + (plsc.cumsum(fgc) - fgc)
        bgrank = bgoff + (plsc.cumsum(bgc) - bgc)
        plsc.store_scatter(fgloc, [jnp.minimum(fgrank, _PER_W - 1)], gidx,
                           mask=fgm)
        plsc.store_scatter(bgloc, [jnp.minimum(bgrank, _PER_W - 1)], gidx,
                           mask=bgm)
        return (fgoff + plsc.all_reduce_population_count(fgm),
                bgoff + plsc.all_reduce_population_count(bgm))

    z = jnp.zeros((16,), jnp.int32)
    fgoff, bgoff = lax.fori_loop(0, _CHUNKS, cbody, (z, z))
    fgcnt = jnp.max(fgoff)
    bgcnt = jnp.max(bgoff)

    cntv[...] = jnp.where(iot == 0, fgcnt, jnp.where(iot == 1, bgcnt, 0))
    pltpu.sync_copy(cntv, cntsh.at[pl.ds(s * 16, 16)])
    plsc.subcore_barrier()
    pltpu.sync_copy(cntsh, callv)
    fgcv = plsc.load_gather(callv, [iot * 16])
    bgcv = plsc.load_gather(callv, [iot * 16 + 1])
    fgstart = plsc.cumsum(jnp.where(iot < s, fgcv, 0))[15]
    bgstart = plsc.cumsum(jnp.where(iot < s, bgcv, 0))[15]
    nfg_c = plsc.cumsum(fgcv)[15]
    nbg_c = plsc.cumsum(bgcv)[15]

    for ch in range(_CHUNKS):
        pos = iot + ch * 16
        tf = jnp.where(pos < fgcnt, fgstart + pos, _TRASH + iot)
        tb = jnp.where(pos < bgcnt, bgstart + pos, _TRASH + iot)
        hsl = pl.ds((ch % 5) * 16, 16)
        if ch < 5:
            tfga[hsl] = tf
            tbga[hsl] = tb
        else:
            tfgb[hsl] = tf
            tbgb[hsl] = tb
    half = _PER_W // 2
    pltpu.sync_copy(fgloc.at[pl.ds(0, half)], fgsh.at[tfga])
    pltpu.sync_copy(fgloc.at[pl.ds(half, half)], fgsh.at[tfgb])
    pltpu.sync_copy(bgloc.at[pl.ds(0, half)], bgsh.at[tbga])
    pltpu.sync_copy(bgloc.at[pl.ds(half, half)], bgsh.at[tbgb])
    plsc.subcore_barrier()

    @pl.when((c == 1) & (s == 0))
    def _():
        c1v[...] = jnp.where(iot == 0, nfg_c, jnp.where(iot == 1, nbg_c, 0))
        pltpu.sync_copy(fgsh.at[pl.ds(0, _XFER)], stagev)
        pltpu.sync_copy(bgsh.at[pl.ds(0, _XFER)], stagev2)
        he1 = pltpu.async_copy(stagev, sth.at[pl.ds(0, _XFER)], sem)
        he2 = pltpu.async_copy(stagev2, sth.at[pl.ds(_XFER, _XFER)], sem)
        he3 = pltpu.async_copy(c1v, sth.at[pl.ds(_ST_CNT, 16)], sem)
        he1.wait()
        he2.wait()
        he3.wait()
        flagv[...] = jnp.full((16,), _MAGIC, jnp.int32)
        pltpu.sync_copy(flagv, sth.at[pl.ds(_ST_FLAG, 16)])

    @pl.when(c == 0)
    def _():
        @pl.when(s == 0)
        def _():
            def spin(i, seen):
                @pl.when(seen == 0)
                def _():
                    pltpu.sync_copy(sth.at[pl.ds(_ST_FLAG, 16)], flagv)
                v = flagv[...]
                return jnp.maximum(seen, jnp.where(v[0] == _MAGIC, 1, 0))

            lax.fori_loop(0, _SPIN, spin, jnp.int32(0))
            hi1 = pltpu.async_copy(sth.at[pl.ds(0, _XFER)], stagev, sem)
            hi2 = pltpu.async_copy(sth.at[pl.ds(_XFER, _XFER)], stagev2, sem)
            hi3 = pltpu.async_copy(sth.at[pl.ds(_ST_CNT, 16)], c1v, sem)
            hi1.wait()
            hi2.wait()
            hi3.wait()
            pltpu.sync_copy(stagev, fg1sh)
            pltpu.sync_copy(stagev2, bg1sh)
            pltpu.sync_copy(c1v, c1sh)

        plsc.subcore_barrier()
        pltpu.sync_copy(c1sh, c1v)
        cv = c1v[...]
        nfg = nfg_c + cv[0]
        nbg = nbg_c + cv[1]

        def sel_idx(h):
            rv = iot + s * 32 + h * 16
            rb = rv - 128
            in_fg = rv < N_FG_OUT
            p_fg = jnp.where(in_fg, rv, rb - nbg)
            p_bg = jnp.where(in_fg, rv - nfg, rb)
            ia = jnp.clip(p_fg, 0, _CAP - 1)
            ib = jnp.clip(p_fg - nfg_c, 0, _XFER - 1)
            ic = jnp.clip(p_bg, 0, _CAP - 1)
            id_ = jnp.clip(p_bg - nbg_c, 0, _XFER - 1)
            return ia, ib, ic, id_

        for h in range(2):
            ia, ib, ic, id_ = sel_idx(h)
            hsl = pl.ds(h * 16, 16)
            sidxa[hsl] = ia
            sidxb[hsl] = ib
            sidxc[hsl] = ic
            sidxd[hsl] = id_
        pltpu.sync_copy(fgsh.at[sidxa], ava)
        pltpu.sync_copy(fg1sh.at[sidxb], avb)
        pltpu.sync_copy(bgsh.at[sidxc], avc)
        pltpu.sync_copy(bg1sh.at[sidxd], avd)

        for h in range(2):
            rv = iot + s * 32 + h * 16
            rb = rv - 128
            in_fg = rv < N_FG_OUT
            use_fgl = jnp.where(in_fg, rv < nfg, rb >= nbg)
            p_fg = jnp.where(in_fg, rv, rb - nbg)
            p_bg = jnp.where(in_fg, rv - nfg, rb)
            hsl = pl.ds(h * 16, 16)
            fgval = jnp.where(p_fg < nfg_c, ava[hsl], avb[hsl])
            bgval = jnp.where(p_bg < nbg_c, avc[hsl], avd[hsl])
            selref[hsl] = jnp.where(use_fgl, fgval, bgval)

        hg1 = pltpu.async_copy(bvh.at[selref], bvres, sem)
        hg2 = pltpu.async_copy(bih.at[selref], bires, sem)
        hg1.wait()
        hg2.wait()

        for h in range(2):
            hsl = pl.ds(h * 16, 16)
            v = bvres[hsl]
            t = bires[hsl]
            ox1[hsl] = plsc.load_gather(gx1, [t])
            oy1[hsl] = plsc.load_gather(gy1, [t])
            ox2[hsl] = plsc.load_gather(gx2, [t])
            oy2[hsl] = plsc.load_gather(gy2, [t])
            oiou[hsl] = v
            ocls[hsl] = jnp.where(v >= 0.5, plsc.load_gather(gcls, [t]),
                                  NUM_CLASSES)

        outs = [pltpu.async_copy(ox1, ox1h.at[osl], sem)
                for ox1, ox1h in ((ox1, ox1h), (oy1, oy1h), (ox2, ox2h),
                                  (oy2, oy2h), (oiou, oiouh), (ocls, oclsh))
                for osl in (pl.ds(s * 32, 32),)]

        @pl.when(s == 0)
        def _():
            cntv[...] = jnp.where(iot == 0, nfg, jnp.where(iot == 1, nbg, 0))
            pltpu.sync_copy(cntv, cnth)

        for h in outs:
            h.wait()


_fused = pl.kernel(
    _fused_body,
    out_type=[jax.ShapeDtypeStruct((512,), jnp.float32)] * 5
    + [jax.ShapeDtypeStruct((512,), jnp.int32),
       jax.ShapeDtypeStruct((16,), jnp.int32),
       jax.ShapeDtypeStruct((N_PAD,), jnp.float32),
       jax.ShapeDtypeStruct((N_PAD,), jnp.int32),
       jax.ShapeDtypeStruct((_ST_LEN,), jnp.int32)],
    mesh=_mesh,
    compiler_params=pltpu.CompilerParams(needs_layout_passes=False),
    scratch_types=[pltpu.VMEM((_PER_W,), jnp.float32)] * 4
    + [pltpu.VMEM((G_PAD,), jnp.float32)] * 4
    + [pltpu.VMEM((G_PAD,), jnp.int32)]
    + [pltpu.VMEM((_PER_W,), jnp.float32),
       pltpu.VMEM((_PER_W,), jnp.int32)]
    + [pltpu.VMEM((_PER_W,), jnp.int32)] * 2
    + [pltpu.VMEM((_PER_W // 2,), jnp.int32)] * 4
    + [pltpu.VMEM((16,), jnp.int32),
       pltpu.VMEM((256,), jnp.int32),
       pltpu.VMEM((16,), jnp.int32)]
    + [pltpu.VMEM((16,), jnp.int32)]
    + [pltpu.VMEM((_XFER,), jnp.int32)] * 2
    + [pltpu.VMEM((32,), jnp.int32)] * 4
    + [pltpu.VMEM((32,), jnp.int32)] * 4
    + [pltpu.VMEM((32,), jnp.int32),
       pltpu.VMEM((32,), jnp.float32),
       pltpu.VMEM((32,), jnp.int32)]
    + [pltpu.VMEM((32,), jnp.float32)] * 5
    + [pltpu.VMEM((32,), jnp.int32)]
    + [pltpu.VMEM_SHARED((_CAP,), jnp.int32)] * 2
    + [pltpu.VMEM_SHARED((256,), jnp.int32)]
    + [pltpu.VMEM_SHARED((_XFER,), jnp.int32)] * 2
    + [pltpu.VMEM_SHARED((16,), jnp.int32)]
    + [pltpu.SemaphoreType.DMA],
)


def kernel(proposal_boxes, gt_boxes, gt_classes):
    boxes = jnp.concatenate([gt_boxes, proposal_boxes], axis=0)
    boxes = jnp.pad(boxes, ((0, N_PAD - N_REAL), (0, 0)))
    gtp = jnp.pad(gt_boxes, ((0, G_PAD - G), (0, 0)))
    gcls = jnp.pad(gt_classes.astype(jnp.int32), (0, G_PAD - G))

    ox1, oy1, ox2, oy2, oiou, ocls, cnt, _, _, _ = _fused(
        boxes[:, 0], boxes[:, 1], boxes[:, 2], boxes[:, 3],
        gtp[:, 0], gtp[:, 1], gtp[:, 2], gtp[:, 3], gcls)

    out = jnp.stack([ox1, oy1, ox2, oy2, oiou], axis=1)
    return out, ocls, cnt[0], cnt[1]


# speedup vs baseline: 1.0732x; 1.0732x over previous
import jax
import jax.numpy as jnp
from jax import lax
from jax.experimental import pallas as pl
from jax.experimental.pallas import tpu as pltpu
from jax.experimental.pallas import tpu_sc as plsc

NUM_CLASSES = 80
N_FG_OUT = 128
N_BG_OUT = 384
N_REAL = 5100
N_PAD = 5120
G = 100
G_PAD = 128

_NC = 2
_NS = 16
_PER_W = N_PAD // (_NC * _NS)
_CHUNKS = _PER_W // 16
_G_CH = 6
_PER_C = N_PAD // _NC

_CAP = _PER_C + 64
_TRASH = _PER_C
_XFER = 384
_ST_LEN = 2 * _XFER + 32
_ST_CNT = 2 * _XFER
_ST_FLAG = 2 * _XFER + 16
_MAGIC = 1234567
_SPIN = 64

_mesh = plsc.VectorSubcoreMesh(core_axis_name="c", subcore_axis_name="s")


def _fused_body(px1h, py1h, px2h, py2h, gx1h, gy1h, gx2h, gy2h, gclsh,
                ox1h, oy1h, ox2h, oy2h, oiouh, oclsh, cnth, bvh, bih, sth,
                px1, py1, px2, py2, gx1, gy1, gx2, gy2, gcls,
                bv, bi, fgloc, bgloc, tfga, tfgb, tbga, tbgb,
                cntv, callv, c1v, flagv, stagev, stagev2,
                sidxa, sidxb, sidxc, sidxd, ava, avb, avc, avd,
                selref, bvres, bires,
                ox1, oy1, ox2, oy2, oiou, ocls,
                fgsh, bgsh, cntsh, fg1sh, bg1sh, c1sh, sem):
    c = lax.axis_index("c")
    s = lax.axis_index("s")
    wid = c * _NS + s
    base = wid * _PER_W
    iot = lax.iota(jnp.int32, 16)

    @pl.when((c == 0) & (s == 0))
    def _():
        c1v[...] = jnp.zeros((16,), jnp.int32)
        pltpu.sync_copy(c1v, sth.at[pl.ds(_ST_FLAG, 16)])

    loads = [pltpu.async_copy(px1h.at[pl.ds(base, _PER_W)], px1, sem),
             pltpu.async_copy(py1h.at[pl.ds(base, _PER_W)], py1, sem),
             pltpu.async_copy(px2h.at[pl.ds(base, _PER_W)], px2, sem),
             pltpu.async_copy(py2h.at[pl.ds(base, _PER_W)], py2, sem),
             pltpu.async_copy(gx1h, gx1, sem),
             pltpu.async_copy(gy1h, gy1, sem),
             pltpu.async_copy(gx2h, gx2, sem),
             pltpu.async_copy(gy2h, gy2, sem),
             pltpu.async_copy(gclsh, gcls, sem)]
    for h in loads:
        h.wait()

    _TILE = 5
    for t in range(_CHUNKS // _TILE):
        sls = [pl.ds((t * _TILE + k) * 16, 16) for k in range(_TILE)]
        x1s = [px1[sl] for sl in sls]
        y1s = [py1[sl] for sl in sls]
        x2s = [px2[sl] for sl in sls]
        y2s = [py2[sl] for sl in sls]
        pas = [(x2s[k] - x1s[k]) * (y2s[k] - y1s[k]) for k in range(_TILE)]

        def body(gc, carry, nl=16, x1s=x1s, y1s=y1s, x2s=x2s, y2s=y2s,
                 pas=pas):
            bvs = list(carry[:_TILE])
            bis = list(carry[_TILE:])
            g0 = pl.ds(gc * 16, 16)
            a1v = gx1[g0]
            b1v = gy1[g0]
            a2v = gx2[g0]
            b2v = gy2[g0]
            gav = (a2v - a1v) * (b2v - b1v)
            jbase = gc * 16
            for l in range(nl):
                a1 = a1v[l]
                b1 = b1v[l]
                a2 = a2v[l]
                b2 = b2v[l]
                ga = gav[l]
                for k in range(_TILE):
                    w = jnp.maximum(
                        jnp.minimum(a2, x2s[k]) - jnp.maximum(a1, x1s[k]),
                        0.0)
                    h = jnp.maximum(
                        jnp.minimum(b2, y2s[k]) - jnp.maximum(b1, y1s[k]),
                        0.0)
                    inter = w * h
                    iou = inter / ((ga + pas[k]) - inter)
                    pred = iou > bvs[k]
                    bvs[k] = jnp.where(pred, iou, bvs[k])
                    bis[k] = jnp.where(pred, jbase + l, bis[k])
            return tuple(bvs) + tuple(bis)

        init = (tuple(jnp.full((16,), -1.0, jnp.float32)
                      for _ in range(_TILE))
                + tuple(jnp.zeros((16,), jnp.int32) for _ in range(_TILE)))
        carry = lax.fori_loop(0, _G_CH, body, init)
        res = body(_G_CH, carry, nl=G - _G_CH * 16)
        for k in range(_TILE):
            bv[sls[k]] = res[k]
            bi[sls[k]] = res[_TILE + k]

    hm1 = pltpu.async_copy(bv, bvh.at[pl.ds(base, _PER_W)], sem)
    hm2 = pltpu.async_copy(bi, bih.at[pl.ds(base, _PER_W)], sem)
    hm1.wait()
    hm2.wait()

    def cbody(ch, carry):
        fgoff, bgoff = carry
        vals = bv[pl.ds(ch * 16, 16)]
        gidx = iot + base + ch * 16
        real = gidx < N_REAL
        fgm = (vals >= 0.5) & real
        bgm = jnp.logical_not(vals >= 0.5) & real
        fgc = jnp.where(fgm, 1, 0)
        bgc = jnp.where(bgm, 1, 0)
        fgrank = fgoff + (plsc.cumsum(fgc) - fgc)
        bgrank = bgoff + (plsc.cumsum(bgc) - bgc)
        plsc.store_scatter(fgloc, [jnp.minimum(fgrank, _PER_W - 1)], gidx,
                           mask=fgm)
        plsc.store_scatter(bgloc, [jnp.minimum(bgrank, _PER_W - 1)], gidx,
                           mask=bgm)
        return (fgoff + plsc.all_reduce_population_count(fgm),
                bgoff + plsc.all_reduce_population_count(bgm))

    z = jnp.zeros((16,), jnp.int32)
    fgoff, bgoff = lax.fori_loop(0, _CHUNKS, cbody, (z, z))
    fgcnt = jnp.max(fgoff)
    bgcnt = jnp.max(bgoff)

    cntv[...] = jnp.where(iot == 0, fgcnt, jnp.where(iot == 1, bgcnt, 0))
    pltpu.sync_copy(cntv, cntsh.at[pl.ds(s * 16, 16)])
    plsc.subcore_barrier()
    pltpu.sync_copy(cntsh, callv)
    fgcv = plsc.load_gather(callv, [iot * 16])
    bgcv = plsc.load_gather(callv, [iot * 16 + 1])
    fgstart = plsc.cumsum(jnp.where(iot < s, fgcv, 0))[15]
    bgstart = plsc.cumsum(jnp.where(iot < s, bgcv, 0))[15]
    nfg_c = plsc.cumsum(fgcv)[15]
    nbg_c = plsc.cumsum(bgcv)[15]

    for ch in range(_CHUNKS):
        pos = iot + ch * 16
        tf = jnp.where(pos < fgcnt, fgstart + pos, _TRASH + iot)
        tb = jnp.where(pos < bgcnt, bgstart + pos, _TRASH + iot)
        hsl = pl.ds((ch % 5) * 16, 16)
        if ch < 5:
            tfga[hsl] = tf
            tbga[hsl] = tb
        else:
            tfgb[hsl] = tf
            tbgb[hsl] = tb
    half = _PER_W // 2
    pltpu.sync_copy(fgloc.at[pl.ds(0, half)], fgsh.at[tfga])
    pltpu.sync_copy(fgloc.at[pl.ds(half, half)], fgsh.at[tfgb])
    pltpu.sync_copy(bgloc.at[pl.ds(0, half)], bgsh.at[tbga])
    pltpu.sync_copy(bgloc.at[pl.ds(half, half)], bgsh.at[tbgb])
    plsc.subcore_barrier()

    @pl.when((c == 1) & (s == 0))
    def _():
        c1v[...] = jnp.where(iot == 0, nfg_c, jnp.where(iot == 1, nbg_c, 0))
        pltpu.sync_copy(fgsh.at[pl.ds(0, _XFER)], stagev)
        pltpu.sync_copy(bgsh.at[pl.ds(0, _XFER)], stagev2)
        he1 = pltpu.async_copy(stagev, sth.at[pl.ds(0, _XFER)], sem)
        he2 = pltpu.async_copy(stagev2, sth.at[pl.ds(_XFER, _XFER)], sem)
        he3 = pltpu.async_copy(c1v, sth.at[pl.ds(_ST_CNT, 16)], sem)
        he1.wait()
        he2.wait()
        he3.wait()
        flagv[...] = jnp.full((16,), _MAGIC, jnp.int32)
        pltpu.sync_copy(flagv, sth.at[pl.ds(_ST_FLAG, 16)])

    @pl.when(c == 0)
    def _():
        @pl.when(s == 0)
        def _():
            def spin(i, seen):
                @pl.when(seen == 0)
                def _():
                    pltpu.sync_copy(sth.at[pl.ds(_ST_FLAG, 16)], flagv)
                v = flagv[...]
                return jnp.maximum(seen, jnp.where(v[0] == _MAGIC, 1, 0))

            lax.fori_loop(0, _SPIN, spin, jnp.int32(0))
            hi1 = pltpu.async_copy(sth.at[pl.ds(0, _XFER)], stagev, sem)
            hi2 = pltpu.async_copy(sth.at[pl.ds(_XFER, _XFER)], stagev2, sem)
            hi3 = pltpu.async_copy(sth.at[pl.ds(_ST_CNT, 16)], c1v, sem)
            hi1.wait()
            hi2.wait()
            hi3.wait()
            pltpu.sync_copy(stagev, fg1sh)
            pltpu.sync_copy(stagev2, bg1sh)
            pltpu.sync_copy(c1v, c1sh)

        plsc.subcore_barrier()
        pltpu.sync_copy(c1sh, c1v)
        cv = c1v[...]
        nfg = nfg_c + cv[0]
        nbg = nbg_c + cv[1]

        def sel_idx(h):
            rv = iot + s * 32 + h * 16
            rb = rv - 128
            in_fg = rv < N_FG_OUT
            p_fg = jnp.where(in_fg, rv, rb - nbg)
            p_bg = jnp.where(in_fg, rv - nfg, rb)
            ia = jnp.clip(p_fg, 0, _CAP - 1)
            ib = jnp.clip(p_fg - nfg_c, 0, _XFER - 1)
            ic = jnp.clip(p_bg, 0, _CAP - 1)
            id_ = jnp.clip(p_bg - nbg_c, 0, _XFER - 1)
            return ia, ib, ic, id_

        for h in range(2):
            ia, ib, ic, id_ = sel_idx(h)
            hsl = pl.ds(h * 16, 16)
            sidxa[hsl] = ia
            sidxb[hsl] = ib
            sidxc[hsl] = ic
            sidxd[hsl] = id_
        pltpu.sync_copy(fgsh.at[sidxa], ava)
        pltpu.sync_copy(fg1sh.at[sidxb], avb)
        pltpu.sync_copy(bgsh.at[sidxc], avc)
        pltpu.sync_copy(bg1sh.at[sidxd], avd)

        for h in range(2):
            rv = iot + s * 32 + h * 16
            rb = rv - 128
            in_fg = rv < N_FG_OUT
            use_fgl = jnp.where(in_fg, rv < nfg, rb >= nbg)
            p_fg = jnp.where(in_fg, rv, rb - nbg)
            p_bg = jnp.where(in_fg, rv - nfg, rb)
            hsl = pl.ds(h * 16, 16)
            fgval = jnp.where(p_fg < nfg_c, ava[hsl], avb[hsl])
            bgval = jnp.where(p_bg < nbg_c, avc[hsl], avd[hsl])
            selref[hsl] = jnp.where(use_fgl, fgval, bgval)

        hg1 = pltpu.async_copy(bvh.at[selref], bvres, sem)
        hg2 = pltpu.async_copy(bih.at[selref], bires, sem)
        hg1.wait()
        hg2.wait()

        for h in range(2):
            hsl = pl.ds(h * 16, 16)
            v = bvres[hsl]
            t = bires[hsl]
            ox1[hsl] = plsc.load_gather(gx1, [t])
            oy1[hsl] = plsc.load_gather(gy1, [t])
            ox2[hsl] = plsc.load_gather(gx2, [t])
            oy2[hsl] = plsc.load_gather(gy2, [t])
            oiou[hsl] = v
            ocls[hsl] = jnp.where(v >= 0.5, plsc.load_gather(gcls, [t]),
                                  NUM_CLASSES)

        outs = [pltpu.async_copy(ox1, ox1h.at[osl], sem)
                for ox1, ox1h in ((ox1, ox1h), (oy1, oy1h), (ox2, ox2h),
                                  (oy2, oy2h), (oiou, oiouh), (ocls, oclsh))
                for osl in (pl.ds(s * 32, 32),)]

        @pl.when(s == 0)
        def _():
            cntv[...] = jnp.where(iot == 0, nfg, jnp.where(iot == 1, nbg, 0))
            pltpu.sync_copy(cntv, cnth)

        for h in outs:
            h.wait()


_fused = pl.kernel(
    _fused_body,
    out_type=[jax.ShapeDtypeStruct((512,), jnp.float32)] * 5
    + [jax.ShapeDtypeStruct((512,), jnp.int32),
       jax.ShapeDtypeStruct((16,), jnp.int32),
       jax.ShapeDtypeStruct((N_PAD,), jnp.float32),
       jax.ShapeDtypeStruct((N_PAD,), jnp.int32),
       jax.ShapeDtypeStruct((_ST_LEN,), jnp.int32)],
    mesh=_mesh,
    compiler_params=pltpu.CompilerParams(needs_layout_passes=False),
    scratch_types=[pltpu.VMEM((_PER_W,), jnp.float32)] * 4
    + [pltpu.VMEM((G_PAD,), jnp.float32)] * 4
    + [pltpu.VMEM((G_PAD,), jnp.int32)]
    + [pltpu.VMEM((_PER_W,), jnp.float32),
       pltpu.VMEM((_PER_W,), jnp.int32)]
    + [pltpu.VMEM((_PER_W,), jnp.int32)] * 2
    + [pltpu.VMEM((_PER_W // 2,), jnp.int32)] * 4
    + [pltpu.VMEM((16,), jnp.int32),
       pltpu.VMEM((256,), jnp.int32),
       pltpu.VMEM((16,), jnp.int32)]
    + [pltpu.VMEM((16,), jnp.int32)]
    + [pltpu.VMEM((_XFER,), jnp.int32)] * 2
    + [pltpu.VMEM((32,), jnp.int32)] * 4
    + [pltpu.VMEM((32,), jnp.int32)] * 4
    + [pltpu.VMEM((32,), jnp.int32),
       pltpu.VMEM((32,), jnp.float32),
       pltpu.VMEM((32,), jnp.int32)]
    + [pltpu.VMEM((32,), jnp.float32)] * 5
    + [pltpu.VMEM((32,), jnp.int32)]
    + [pltpu.VMEM_SHARED((_CAP,), jnp.int32)] * 2
    + [pltpu.VMEM_SHARED((256,), jnp.int32)]
    + [pltpu.VMEM_SHARED((_XFER,), jnp.int32)] * 2
    + [pltpu.VMEM_SHARED((16,), jnp.int32)]
    + [pltpu.SemaphoreType.DMA],
)


def kernel(proposal_boxes, gt_boxes, gt_classes):
    boxes = jnp.concatenate([gt_boxes, proposal_boxes], axis=0)
    boxes = jnp.pad(boxes, ((0, N_PAD - N_REAL), (0, 0)))
    gtp = jnp.pad(gt_boxes, ((0, G_PAD - G), (0, 0)))
    gcls = jnp.pad(gt_classes.astype(jnp.int32), (0, G_PAD - G))

    ox1, oy1, ox2, oy2, oiou, ocls, cnt, _, _, _ = _fused(
        boxes[:, 0], boxes[:, 1], boxes[:, 2], boxes[:, 3],
        gtp[:, 0], gtp[:, 1], gtp[:, 2], gtp[:, 3], gcls)

    out = jnp.stack([ox1, oy1, ox2, oy2, oiou], axis=1)
    return out, ocls, cnt[0], cnt[1]
